# baseline (speedup 1.0000x reference)
import functools

import jax
import jax.numpy as jnp
from jax import lax
from jax.experimental import pallas as pl
from jax.experimental.pallas import tpu as pltpu
from jax.experimental.pallas import tpu_sc as plsc

D_MODEL = 64
SCALE = 8.0
_L = 16
_NC = 2
_NS = 16
_NW = _NC * _NS
_HALVES = (104, 96)


@functools.lru_cache(maxsize=None)
def _make_kernel(R, C, V):
    rpw = R // _NW
    mesh = plsc.VectorSubcoreMesh(core_axis_name="c", subcore_axis_name="s")

    @functools.partial(
        pl.kernel,
        mesh=mesh,
        out_type=jax.ShapeDtypeStruct((R, C, D_MODEL), jnp.float32),
        scratch_types=[
            pltpu.VMEM((rpw, C), jnp.int32),
            pltpu.VMEM((max(_HALVES), D_MODEL), jnp.float32),
            pltpu.SemaphoreType.DMA,
        ],
        compiler_params=pltpu.CompilerParams(use_tc_tiling_on_sc=False),
    )
    def k(x_hbm, lut_hbm, out_hbm, idx_v, rows_v, sem):
        wid = lax.axis_index("s") * _NC + lax.axis_index("c")
        base = wid * rpw
        pltpu.sync_copy(x_hbm.at[pl.ds(base, rpw)], idx_v)

        def row_body(r, carry):
            off = 0
            for n in _HALVES:
                idx_ref = idx_v.at[r, pl.ds(off, n)]
                dst = rows_v.at[pl.ds(0, n)]
                pltpu.async_copy(lut_hbm.at[idx_ref], dst, sem).wait()

                def scale_body(i, carry2):
                    for j in range(D_MODEL // _L):
                        sl = pl.ds(j * _L, _L)
                        rows_v[i, sl] = rows_v[i, sl] * SCALE
                    return carry2

                lax.fori_loop(0, n, scale_body, 0)
                pltpu.sync_copy(dst, out_hbm.at[base + r, pl.ds(off, n)])
                off += n
            return carry

        lax.fori_loop(0, rpw, row_body, 0)

    return k


def kernel(x, lut):
    out = _make_kernel(x.shape[0], x.shape[1], lut.shape[0])(
        x.astype(jnp.int32), lut
    )
    return out


# speedup vs baseline: 1.4070x; 1.4070x over previous
import functools

import jax
import jax.numpy as jnp
from jax import lax
from jax.experimental import pallas as pl
from jax.experimental.pallas import tpu as pltpu
from jax.experimental.pallas import tpu_sc as plsc
from jax.experimental.layout import Layout, Format, with_layout_constraint

D_MODEL = 64
SCALE = 8.0
_L = 16
_NC = 2
_NS = 16
_NW = _NC * _NS


@functools.lru_cache(maxsize=None)
def _make_kernel(R, C, V):
    rpw = R // _NW
    mesh = plsc.VectorSubcoreMesh(core_axis_name="c", subcore_axis_name="s")

    @functools.partial(
        pl.kernel,
        mesh=mesh,
        out_type=jax.ShapeDtypeStruct((R, C, D_MODEL), jnp.float32),
        scratch_types=[
            pltpu.VMEM((C, rpw), jnp.int32),
            pltpu.VMEM((2, rpw, D_MODEL), jnp.float32),
            pltpu.VMEM((2, rpw, 1, D_MODEL), jnp.float32),
            pltpu.SemaphoreType.DMA,
            pltpu.SemaphoreType.DMA,
            pltpu.SemaphoreType.DMA,
            pltpu.SemaphoreType.DMA,
        ],
        compiler_params=pltpu.CompilerParams(use_tc_tiling_on_sc=False),
    )
    def k(xt_hbm, lut_hbm, out_hbm, idx_v, rows_v, srows_v, g0, g1, s0, s1):
        wid = lax.axis_index("s") * _NC + lax.axis_index("c")
        base = wid * rpw
        pltpu.sync_copy(xt_hbm.at[:, pl.ds(base, rpw)], idx_v)
        gsem = (g0, g1)
        ssem = (s0, s1)

        def gather_copy(c, b):
            return pltpu.make_async_copy(
                lut_hbm.at[idx_v.at[c]], rows_v.at[b], gsem[b]
            )

        def store_copy(c, b):
            return pltpu.make_async_copy(
                srows_v.at[b], out_hbm.at[pl.ds(base, rpw), pl.ds(c, 1)], ssem[b]
            )

        gather_copy(0, 0).start()
        gather_copy(1, 1).start()

        def outer(o, carry):
            for b in range(2):
                c = o * 2 + b
                gather_copy(c, b).wait()

                @pl.when(c >= 2)
                def _():
                    store_copy(c - 2, b).wait()

                def scale_body(i, carry2):
                    for j in range(D_MODEL // _L):
                        sl = pl.ds(j * _L, _L)
                        srows_v[b, i, 0, sl] = rows_v[b, i, sl] * SCALE
                    return carry2

                lax.fori_loop(0, rpw, scale_body, 0)
                store_copy(c, b).start()

                @pl.when(c + 2 < C)
                def _():
                    gather_copy(c + 2, b).start()

            return carry

        lax.fori_loop(0, C // 2, outer, 0)
        store_copy(C - 2, 0).wait()
        store_copy(C - 1, 1).wait()

    return k


def kernel(x, lut):
    xt = x.T.astype(jnp.int32)
    out = _make_kernel(x.shape[0], x.shape[1], lut.shape[0])(xt, lut)
    return with_layout_constraint(
        out, Layout(major_to_minor=(0, 1, 2), tiling=())
    )
